# Initial kernel scaffold; baseline (speedup 1.0000x reference)
import jax
import jax.numpy as jnp
from jax.experimental import pallas as pl


def kernel(x, wte):
    raise NotImplementedError("write your pallas kernel here")


# speedup vs baseline: 1.7796x; 1.7796x over previous
import functools

import jax
import jax.numpy as jnp
from jax import lax
from jax.experimental import pallas as pl
from jax.experimental.pallas import tpu as pltpu
from jax.experimental.pallas import tpu_sc as plsc


def _make_emb_kernel(B, V, D, NC, NS):
    NW = NC * NS
    BPW = B // NW
    C = 16
    NCHUNK = BPW // C
    mesh = plsc.VectorSubcoreMesh(core_axis_name="c", subcore_axis_name="s")

    @functools.partial(
        pl.kernel,
        mesh=mesh,
        out_type=jax.ShapeDtypeStruct((B, D), jnp.float32),
        scratch_types=[
            pltpu.VMEM((BPW,), jnp.int32),
            pltpu.VMEM((C, D), jnp.float32),
            pltpu.VMEM((C, D), jnp.float32),
            pltpu.SemaphoreType.DMA,
            pltpu.SemaphoreType.DMA,
        ],
    )
    def emb(idx_hbm, table_hbm, out_hbm, idx_v, buf0, buf1, sem0, sem1):
        wid = lax.axis_index("s") * NC + lax.axis_index("c")
        base = wid * BPW
        pltpu.sync_copy(idx_hbm.at[pl.ds(base, BPW)], idx_v)

        bufs = (buf0, buf1)
        sems = (sem0, sem1)

        pltpu.async_copy(table_hbm.at[idx_v.at[pl.ds(0, C)]], buf0, sem0)

        def body(g, carry):
            for p in range(2):
                @pl.when(lax.rem(g, 2) == p)
                def _():
                    cur, nxt = bufs[p], bufs[1 - p]
                    csem, nsem = sems[p], sems[1 - p]
                    @pl.when(g + 1 < NCHUNK)
                    def _():
                        pltpu.async_copy(
                            table_hbm.at[idx_v.at[pl.ds((g + 1) * C, C)]],
                            nxt, nsem)
                    pltpu.make_async_copy(
                        table_hbm.at[idx_v.at[pl.ds(g * C, C)]], cur, csem
                    ).wait()
                    pltpu.sync_copy(cur, out_hbm.at[pl.ds(base + g * C, C)])
            return carry

        lax.fori_loop(0, NCHUNK, body, 0)

    return emb


def kernel(x, wte):
    B = x.size
    V, D = wte.shape
    info = plsc.get_sparse_core_info()
    emb = _make_emb_kernel(B, V, D, info.num_cores, info.num_subcores)
    out = emb(x.reshape(B).astype(jnp.int32), wte)
    return out.reshape(x.shape + (D,))


# speedup vs baseline: 1.7842x; 1.0026x over previous
import functools

import jax
import jax.numpy as jnp
from jax import lax
from jax.experimental import pallas as pl
from jax.experimental.pallas import tpu as pltpu
from jax.experimental.pallas import tpu_sc as plsc

_NBUF = 3


def _make_emb_kernel(B, V, D, NC, NS):
    NW = NC * NS
    BPW = B // NW
    C = 16
    NCHUNK = BPW // C
    mesh = plsc.VectorSubcoreMesh(core_axis_name="c", subcore_axis_name="s")

    @functools.partial(
        pl.kernel,
        mesh=mesh,
        out_type=jax.ShapeDtypeStruct((B, D), jnp.float32),
        scratch_types=[
            pltpu.VMEM((BPW,), jnp.int32),
            pltpu.VMEM((C, D), jnp.float32),
            pltpu.VMEM((C, D), jnp.float32),
            pltpu.VMEM((C, D), jnp.float32),
            pltpu.SemaphoreType.DMA,
            pltpu.SemaphoreType.DMA,
            pltpu.SemaphoreType.DMA,
            pltpu.SemaphoreType.DMA,
            pltpu.SemaphoreType.DMA,
            pltpu.SemaphoreType.DMA,
        ],
    )
    def emb(idx_hbm, table_hbm, out_hbm, idx_v,
            b0, b1, b2, g0, g1, g2, w0, w1, w2):
        bufs = (b0, b1, b2)
        gsems = (g0, g1, g2)
        wsems = (w0, w1, w2)
        wid = lax.axis_index("s") * NC + lax.axis_index("c")
        base = wid * BPW
        pltpu.sync_copy(idx_hbm.at[pl.ds(base, BPW)], idx_v)

        def gather(chunk, p):
            pltpu.async_copy(
                table_hbm.at[idx_v.at[pl.ds(chunk * C, C)]], bufs[p], gsems[p])

        gather(0, 0)
        gather(1, 1)

        def body(g, carry):
            for p in range(_NBUF):
                q = (p + 2) % _NBUF

                @pl.when(lax.rem(g, _NBUF) == p)
                def _(p=p, q=q):
                    @pl.when(jnp.logical_and(g + 2 < NCHUNK, g >= 1))
                    def _():
                        pltpu.make_async_copy(
                            bufs[q],
                            out_hbm.at[pl.ds(base + (g - 1) * C, C)],
                            wsems[q]).wait()

                    @pl.when(g + 2 < NCHUNK)
                    def _():
                        gather(g + 2, q)

                    pltpu.make_async_copy(
                        table_hbm.at[idx_v.at[pl.ds(g * C, C)]],
                        bufs[p], gsems[p]).wait()
                    pltpu.async_copy(
                        bufs[p], out_hbm.at[pl.ds(base + g * C, C)], wsems[p])
            return carry

        lax.fori_loop(0, NCHUNK, body, 0)

        for p in range(_NBUF):
            pltpu.make_async_copy(
                bufs[p], out_hbm.at[pl.ds(base, C)], wsems[p]).wait()

    return emb


def kernel(x, wte):
    B = x.size
    V, D = wte.shape
    info = plsc.get_sparse_core_info()
    emb = _make_emb_kernel(B, V, D, info.num_cores, info.num_subcores)
    out = emb(x.reshape(B).astype(jnp.int32), wte)
    return out.reshape(x.shape + (D,))
